# baseline (speedup 1.0000x reference)
import functools

import jax
import jax.numpy as jnp
from jax import lax
from jax.experimental import pallas as pl
from jax.experimental.pallas import tpu as pltpu
from jax.experimental.pallas import tpu_sc as plsc

HEADS = 8
DIM_HEAD = 64
N_POINTS = 8
DIM = 768
INNER = HEADS * DIM_HEAD
GRID = 32
HW = GRID * GRID
B = 4
N = 64
HP = HEADS * N_POINTS


def _gelu_exact(x):
    return 0.5 * x * (1.0 + lax.erf(x * (2.0 ** -0.5)))


def _stage1(xT_ref, xoT_ref, ctx_ref, Wv_ref, oW1T_ref, ob1_ref, oW2xT_ref,
            oW2yT_ref, ob2x_ref, ob2y_ref, aW1T_ref, ab1_ref, aW2T_ref,
            ab2_ref, vt_ref, idx_ref, w_ref):
    b = pl.program_id(0)
    xT = xT_ref[0]
    xoT = xoT_ref[0]
    ctx = ctx_ref[0]

    v = jnp.dot(ctx, Wv_ref[...], preferred_element_type=jnp.float32)
    for h in range(HEADS):
        vh = v[:, h * DIM_HEAD:(h + 1) * DIM_HEAD]
        vt_ref[h * HW:(h + 1) * HW, 0:DIM_HEAD] = vh
        vt_ref[h * HW:(h + 1) * HW, DIM_HEAD:2 * DIM_HEAD] = jnp.concatenate(
            [vh[1:], vh[HW - 1:HW]], axis=0)

    h_aw = _gelu_exact(jnp.dot(aW1T_ref[...], xT,
                               preferred_element_type=jnp.float32) + ab1_ref[...])
    logits = jnp.dot(aW2T_ref[...], h_aw,
                     preferred_element_type=jnp.float32) + ab2_ref[...]
    e = jnp.exp(logits)
    ri = lax.broadcasted_iota(jnp.int32, (HP, HP), 0) // N_POINTS
    rj = lax.broadcasted_iota(jnp.int32, (HP, HP), 1) // N_POINTS
    S = (ri == rj).astype(jnp.float32)
    attw = e / jnp.dot(S, e, preferred_element_type=jnp.float32)

    h_off = _gelu_exact(jnp.dot(oW1T_ref[...], xoT,
                                preferred_element_type=jnp.float32) + ob1_ref[...])
    gx = jnp.tanh(jnp.dot(oW2xT_ref[...], h_off,
                          preferred_element_type=jnp.float32) + ob2x_ref[...])
    gy = jnp.tanh(jnp.dot(oW2yT_ref[...], h_off,
                          preferred_element_type=jnp.float32) + ob2y_ref[...])

    half = (GRID - 1) * 0.5
    ix = (gx + 1.0) * half
    iy = (gy + 1.0) * half
    ix0f = jnp.floor(ix)
    iy0f = jnp.floor(iy)
    wx1 = ix - ix0f
    wx0 = 1.0 - wx1
    wy1 = iy - iy0f
    wy0 = 1.0 - wy1
    ix0 = ix0f.astype(jnp.int32)
    iy0 = iy0f.astype(jnp.int32)
    iy1 = jnp.minimum(iy0 + 1, GRID - 1)

    hrow = lax.broadcasted_iota(jnp.int32, (HP, N), 0) // N_POINTS
    mapbase = (b * HEADS + hrow) * HW

    for c, (cy, cwy) in enumerate(((iy0, wy0), (iy1, wy1))):
        idx_ref[0, c] = mapbase + cy * GRID + ix0
        w_ref[0, c, 0] = attw * cwy * wx0
        w_ref[0, c, 1] = attw * cwy * wx1


def _sc_body(table, idx_hbm, w_hbm, out_hbm, idx_v, w2_v, w_v, G_v, acc_v,
             sem):
    c = lax.axis_index("c")
    s = lax.axis_index("s")
    wid = c * 16 + s
    b = wid // HEADS
    h = wid % HEADS

    for corner in range(2):
        pltpu.sync_copy(idx_hbm.at[b, corner, pl.ds(h * N_POINTS, N_POINTS)],
                        idx_v.at[pl.ds(corner * N_POINTS, N_POINTS)])
        for y in range(2):
            blk = corner * 2 + y
            pltpu.sync_copy(
                w_hbm.at[b, corner, y, pl.ds(h * N_POINTS, N_POINTS)],
                w2_v.at[pl.ds(blk * N_POINTS, N_POINTS)])
    for j in range(4 * N_POINTS):
        for k in range(4):
            w_v[pl.ds(j * N + k * 16, 16)] = w2_v[j, k * 16:(k + 1) * 16]

    for jc in range(2):
        copies = []
        for j in range(N_POINTS):
            copies.append(pltpu.make_async_copy(
                table.at[idx_v.at[jc * N_POINTS + j]],
                G_v.at[pl.ds(j * N, N)], sem))
        for cp in copies:
            cp.start()
        for cp in copies:
            cp.wait()

        def nbody(n, _, jc=jc):
            nvec = jnp.full((16,), n, dtype=jnp.int32)
            accs = [None] * 4
            for j in range(N_POINTS):
                wbA = plsc.load_gather(
                    w_v, [nvec + ((jc * 2 + 0) * N_POINTS + j) * N])
                wbB = plsc.load_gather(
                    w_v, [nvec + ((jc * 2 + 1) * N_POINTS + j) * N])
                row = j * N + n
                for k in range(4):
                    gA = G_v[row, k * 16:(k + 1) * 16]
                    gB = G_v[row, DIM_HEAD + k * 16:DIM_HEAD + (k + 1) * 16]
                    t = wbA * gA + wbB * gB
                    accs[k] = t if accs[k] is None else accs[k] + t
            for k in range(4):
                if jc == 0:
                    acc_v[n, k * 16:(k + 1) * 16] = accs[k]
                else:
                    acc_v[n, k * 16:(k + 1) * 16] = (
                        acc_v[n, k * 16:(k + 1) * 16] + accs[k])
            return 0

        lax.fori_loop(0, N, nbody, 0)

    pltpu.sync_copy(acc_v, out_hbm.at[b, h])


def _stage3(s_ref, pW_ref, pb_ref, out_ref):
    acc = None
    for h in range(HEADS):
        t = jnp.dot(s_ref[0, h], pW_ref[h * DIM_HEAD:(h + 1) * DIM_HEAD],
                    preferred_element_type=jnp.float32)
        acc = t if acc is None else acc + t
    out_ref[0] = acc + pb_ref[...]


@jax.jit
def kernel(x, context, W_q, W_v, off_W1, off_b1, off_W2, off_b2,
           aw_W1, aw_b1, aw_W2, aw_b2, out_W, out_b):
    x_perm = jnp.transpose(x.reshape(4, 16, 4, DIM), (2, 0, 1, 3)).reshape(
        B, N, DIM)
    xT = jnp.transpose(x, (0, 2, 1))
    xoT = jnp.transpose(x_perm, (0, 2, 1))

    full = lambda *s: pl.BlockSpec(s, lambda b: (0,) * len(s))
    col = lambda v: v.reshape(-1, 1)

    vt, idx, w = pl.pallas_call(
        _stage1,
        grid=(B,),
        in_specs=[
            pl.BlockSpec((1, DIM, N), lambda b: (b, 0, 0)),
            pl.BlockSpec((1, DIM, N), lambda b: (b, 0, 0)),
            pl.BlockSpec((1, HW, DIM), lambda b: (b, 0, 0)),
            full(DIM, INNER),
            full(DIM, DIM),
            full(DIM, 1),
            full(HP, DIM),
            full(HP, DIM),
            full(HP, 1),
            full(HP, 1),
            full(DIM, DIM),
            full(DIM, 1),
            full(HP, DIM),
            full(HP, 1),
        ],
        out_specs=[
            pl.BlockSpec((HEADS * HW, 2 * DIM_HEAD), lambda b: (b, 0)),
            pl.BlockSpec((1, 2, HP, N), lambda b: (b, 0, 0, 0)),
            pl.BlockSpec((1, 2, 2, HP, N), lambda b: (b, 0, 0, 0, 0)),
        ],
        out_shape=[
            jax.ShapeDtypeStruct((B * HEADS * HW, 2 * DIM_HEAD), jnp.float32),
            jax.ShapeDtypeStruct((B, 2, HP, N), jnp.int32),
            jax.ShapeDtypeStruct((B, 2, 2, HP, N), jnp.float32),
        ],
    )(xT, xoT, context, W_v,
      jnp.transpose(off_W1), col(off_b1),
      jnp.transpose(off_W2[:, 0::2]), jnp.transpose(off_W2[:, 1::2]),
      col(off_b2[0::2]), col(off_b2[1::2]),
      jnp.transpose(aw_W1), col(aw_b1),
      jnp.transpose(aw_W2), col(aw_b2))

    mesh = plsc.VectorSubcoreMesh(core_axis_name="c", subcore_axis_name="s")
    sampled = pl.kernel(
        _sc_body,
        out_type=jax.ShapeDtypeStruct((B, HEADS, N, DIM_HEAD), jnp.float32),
        mesh=mesh,
        compiler_params=pltpu.CompilerParams(needs_layout_passes=False,
                                             use_tc_tiling_on_sc=True),
        scratch_types=[
            pltpu.VMEM((2 * N_POINTS, N), jnp.int32),
            pltpu.VMEM((4 * N_POINTS, N), jnp.float32),
            pltpu.VMEM((4 * N_POINTS * N,), jnp.float32),
            pltpu.VMEM((N_POINTS * N, 2 * DIM_HEAD), jnp.float32),
            pltpu.VMEM((N, DIM_HEAD), jnp.float32),
            pltpu.SemaphoreType.DMA,
        ],
    )(vt, idx, w)

    out = pl.pallas_call(
        _stage3,
        grid=(B,),
        in_specs=[
            pl.BlockSpec((1, HEADS, N, DIM_HEAD), lambda b: (b, 0, 0, 0)),
            full(INNER, DIM),
            full(1, DIM),
        ],
        out_specs=pl.BlockSpec((1, N, DIM), lambda b: (b, 0, 0)),
        out_shape=jax.ShapeDtypeStruct((B, N, DIM), jnp.float32),
    )(sampled, out_W, out_b.reshape(1, -1))
    return out


# speedup vs baseline: 1.4211x; 1.4211x over previous
import functools

import jax
import jax.numpy as jnp
from jax import lax
from jax.experimental import pallas as pl
from jax.experimental.pallas import tpu as pltpu
from jax.experimental.pallas import tpu_sc as plsc

HEADS = 8
DIM_HEAD = 64
N_POINTS = 8
DIM = 768
INNER = HEADS * DIM_HEAD
GRID = 32
HW = GRID * GRID
B = 4
N = 64
HP = HEADS * N_POINTS


def _gelu_exact(x):
    return 0.5 * x * (1.0 + lax.erf(x * (2.0 ** -0.5)))


def _dgT(lhs, rhs, precision=None):
    return lax.dot_general(lhs, rhs, (((0,), (0,)), ((), ())),
                           precision=precision,
                           preferred_element_type=jnp.float32)


def _stage1(x_ref, ctx_ref, Wv_ref, oW1_ref, ob1_ref, oW2_ref, ob2_ref,
            aW1_ref, ab1_ref, aW2_ref, ab2_ref, vt_ref, idx_ref, w_ref):
    b = pl.program_id(0)
    xall = x_ref[...]
    ctx = ctx_ref[0]

    v = jnp.dot(ctx, Wv_ref[...], preferred_element_type=jnp.float32)
    for h in range(HEADS):
        vh = v[:, h * DIM_HEAD:(h + 1) * DIM_HEAD]
        vt_ref[h * HW:(h + 1) * HW, 0:DIM_HEAD] = vh
        vt_ref[h * HW:(h + 1) * HW, DIM_HEAD:2 * DIM_HEAD] = jnp.concatenate(
            [vh[1:], vh[HW - 1:HW]], axis=0)

    x_b = x_ref[pl.ds(b * N, N), :]
    h_aw = _gelu_exact(
        lax.dot_general(aW1_ref[...], x_b, (((0,), (1,)), ((), ())),
                        preferred_element_type=jnp.float32) + ab1_ref[...])
    logits = _dgT(aW2_ref[...], h_aw,
                  precision=lax.Precision.HIGHEST) + ab2_ref[...]
    e = jnp.exp(logits)
    ri = lax.broadcasted_iota(jnp.int32, (HP, HP), 0) // N_POINTS
    rj = lax.broadcasted_iota(jnp.int32, (HP, HP), 1) // N_POINTS
    S = (ri == rj).astype(jnp.float32)
    attw = e / jnp.dot(S, e, preferred_element_type=jnp.float32)

    ni = lax.broadcasted_iota(jnp.int32, (N, B * N), 0)
    rsel = (ni // 16) * N + (ni % 16) * 4 + b
    Gsel = (rsel == lax.broadcasted_iota(jnp.int32, (N, B * N), 1)
            ).astype(jnp.float32)
    xo_b = jnp.dot(Gsel, xall, precision=lax.Precision.HIGHEST,
                   preferred_element_type=jnp.float32)
    h_off = _gelu_exact(
        lax.dot_general(oW1_ref[...], xo_b, (((0,), (1,)), ((), ())),
                        preferred_element_type=jnp.float32) + ob1_ref[...])
    t2 = jnp.tanh(_dgT(oW2_ref[...], h_off,
                       precision=lax.Precision.HIGHEST) + ob2_ref[...])
    pi = lax.broadcasted_iota(jnp.int32, (HP, 2 * HP), 0)
    pj = lax.broadcasted_iota(jnp.int32, (HP, 2 * HP), 1)
    gx = jnp.dot((pj == 2 * pi).astype(jnp.float32), t2,
                 precision=lax.Precision.HIGHEST,
                 preferred_element_type=jnp.float32)
    gy = jnp.dot((pj == 2 * pi + 1).astype(jnp.float32), t2,
                 precision=lax.Precision.HIGHEST,
                 preferred_element_type=jnp.float32)

    half = (GRID - 1) * 0.5
    ix = (gx + 1.0) * half
    iy = (gy + 1.0) * half
    ix0f = jnp.floor(ix)
    iy0f = jnp.floor(iy)
    wx1 = ix - ix0f
    wx0 = 1.0 - wx1
    wy1 = iy - iy0f
    wy0 = 1.0 - wy1
    ix0 = ix0f.astype(jnp.int32)
    iy0 = iy0f.astype(jnp.int32)
    iy1 = jnp.minimum(iy0 + 1, GRID - 1)

    hrow = lax.broadcasted_iota(jnp.int32, (HP, N), 0) // N_POINTS
    mapbase = (b * HEADS + hrow) * HW

    for c, (cy, cwy) in enumerate(((iy0, wy0), (iy1, wy1))):
        idx_ref[0, c] = mapbase + cy * GRID + ix0
        w_ref[0, c, 0] = attw * cwy * wx0
        w_ref[0, c, 1] = attw * cwy * wx1


def _sc_body(table, idx_hbm, w_hbm, out_hbm, idx_v, w2_v, w_v, G_v, acc_v,
             sem, gsem0, gsem1):
    gsems = (gsem0, gsem1)
    c = lax.axis_index("c")
    s = lax.axis_index("s")
    wid = c * 16 + s
    b = wid // HEADS
    h = wid % HEADS

    stage = []
    for corner in range(2):
        stage.append(pltpu.make_async_copy(
            idx_hbm.at[b, corner, pl.ds(h * N_POINTS, N_POINTS)],
            idx_v.at[pl.ds(corner * N_POINTS, N_POINTS)], sem))
        for y in range(2):
            blk = corner * 2 + y
            stage.append(pltpu.make_async_copy(
                w_hbm.at[b, corner, y, pl.ds(h * N_POINTS, N_POINTS)],
                w2_v.at[pl.ds(blk * N_POINTS, N_POINTS)], sem))
    for cp in stage:
        cp.start()
    for cp in stage:
        cp.wait()
    for j in range(4 * N_POINTS):
        for k in range(4):
            w_v[pl.ds(j * N + k * 16, 16)] = w2_v[j, k * 16:(k + 1) * 16]

    NR = 4
    ROUNDS = 16 // NR

    def fire(r, buf):
        cps = []
        for j in range(NR):
            cps.append(pltpu.make_async_copy(
                table.at[idx_v.at[r * NR + j]],
                G_v.at[pl.ds((buf * NR + j) * N, N)], gsems[buf]))
        for cp in cps:
            cp.start()
        return cps

    pend = fire(0, 0)
    for r in range(ROUNDS):
        nxt = fire(r + 1, (r + 1) % 2) if r + 1 < ROUNDS else None
        for cp in pend:
            cp.wait()
        buf = r % 2

        def nbody(n, _, r=r, buf=buf):
            nvec = jnp.full((16,), n, dtype=jnp.int32)
            accs = [None] * 4
            for j in range(NR):
                jj = r * NR + j
                cnr = jj // N_POINTS
                p = jj % N_POINTS
                wbA = plsc.load_gather(
                    w_v, [nvec + ((cnr * 2 + 0) * N_POINTS + p) * N])
                wbB = plsc.load_gather(
                    w_v, [nvec + ((cnr * 2 + 1) * N_POINTS + p) * N])
                row = (buf * NR + j) * N + n
                for k in range(4):
                    gA = G_v[row, k * 16:(k + 1) * 16]
                    gB = G_v[row, DIM_HEAD + k * 16:DIM_HEAD + (k + 1) * 16]
                    t = wbA * gA + wbB * gB
                    accs[k] = t if accs[k] is None else accs[k] + t
            for k in range(4):
                if r == 0:
                    acc_v[n, k * 16:(k + 1) * 16] = accs[k]
                else:
                    acc_v[n, k * 16:(k + 1) * 16] = (
                        acc_v[n, k * 16:(k + 1) * 16] + accs[k])
            return 0

        lax.fori_loop(0, N, nbody, 0)
        pend = nxt

    pltpu.sync_copy(acc_v, out_hbm.at[b, h])


def _stage3(s_ref, pW_ref, pb_ref, out_ref):
    acc = None
    for h in range(HEADS):
        t = jnp.dot(s_ref[0, h], pW_ref[h * DIM_HEAD:(h + 1) * DIM_HEAD],
                    preferred_element_type=jnp.float32)
        acc = t if acc is None else acc + t
    out_ref[0] = acc + pb_ref[...]


@jax.jit
def kernel(x, context, W_q, W_v, off_W1, off_b1, off_W2, off_b2,
           aw_W1, aw_b1, aw_W2, aw_b2, out_W, out_b):
    full = lambda *s: pl.BlockSpec(s, lambda b: (0,) * len(s))
    col = lambda v: v.reshape(-1, 1)

    vt, idx, w = pl.pallas_call(
        _stage1,
        grid=(B,),
        in_specs=[
            full(B * N, DIM),
            pl.BlockSpec((1, HW, DIM), lambda b: (b, 0, 0)),
            full(DIM, INNER),
            full(DIM, DIM),
            full(DIM, 1),
            full(DIM, 2 * HP),
            full(2 * HP, 1),
            full(DIM, DIM),
            full(DIM, 1),
            full(DIM, HP),
            full(HP, 1),
        ],
        out_specs=[
            pl.BlockSpec((HEADS * HW, 2 * DIM_HEAD), lambda b: (b, 0)),
            pl.BlockSpec((1, 2, HP, N), lambda b: (b, 0, 0, 0)),
            pl.BlockSpec((1, 2, 2, HP, N), lambda b: (b, 0, 0, 0, 0)),
        ],
        out_shape=[
            jax.ShapeDtypeStruct((B * HEADS * HW, 2 * DIM_HEAD), jnp.float32),
            jax.ShapeDtypeStruct((B, 2, HP, N), jnp.int32),
            jax.ShapeDtypeStruct((B, 2, 2, HP, N), jnp.float32),
        ],
    )(x.reshape(B * N, DIM), context, W_v,
      off_W1, col(off_b1), off_W2, col(off_b2),
      aw_W1, col(aw_b1), aw_W2, col(aw_b2))

    mesh = plsc.VectorSubcoreMesh(core_axis_name="c", subcore_axis_name="s")
    sampled = pl.kernel(
        _sc_body,
        out_type=jax.ShapeDtypeStruct((B, HEADS, N, DIM_HEAD), jnp.float32),
        mesh=mesh,
        compiler_params=pltpu.CompilerParams(needs_layout_passes=False,
                                             use_tc_tiling_on_sc=True),
        scratch_types=[
            pltpu.VMEM((2 * N_POINTS, N), jnp.int32),
            pltpu.VMEM((4 * N_POINTS, N), jnp.float32),
            pltpu.VMEM((4 * N_POINTS * N,), jnp.float32),
            pltpu.VMEM((8 * N, 2 * DIM_HEAD), jnp.float32),
            pltpu.VMEM((N, DIM_HEAD), jnp.float32),
            pltpu.SemaphoreType.DMA,
            pltpu.SemaphoreType.DMA,
            pltpu.SemaphoreType.DMA,
        ],
    )(vt, idx, w)

    out = pl.pallas_call(
        _stage3,
        grid=(B,),
        in_specs=[
            pl.BlockSpec((1, HEADS, N, DIM_HEAD), lambda b: (b, 0, 0, 0)),
            full(INNER, DIM),
            full(1, DIM),
        ],
        out_specs=pl.BlockSpec((1, N, DIM), lambda b: (b, 0, 0)),
        out_shape=jax.ShapeDtypeStruct((B, N, DIM), jnp.float32),
    )(sampled, out_W, out_b.reshape(1, -1))
    return out


# speedup vs baseline: 1.6465x; 1.1586x over previous
import functools

import jax
import jax.numpy as jnp
from jax import lax
from jax.experimental import pallas as pl
from jax.experimental.pallas import tpu as pltpu
from jax.experimental.pallas import tpu_sc as plsc

HEADS = 8
DIM_HEAD = 64
N_POINTS = 8
DIM = 768
INNER = HEADS * DIM_HEAD
GRID = 32
HW = GRID * GRID
B = 4
N = 64
HP = HEADS * N_POINTS


def _gelu_exact(x):
    return 0.5 * x * (1.0 + lax.erf(x * (2.0 ** -0.5)))


def _stage1(x_ref, ctx_ref, Wv_ref, oW1_ref, ob1_ref, oW2_ref, ob2_ref,
            aW1_ref, ab1_ref, aW2_ref, ab2_ref, vt_ref, idx_ref, w_ref):
    b = pl.program_id(0)
    xall = x_ref[...]
    ctx = ctx_ref[0]

    v = jnp.dot(ctx, Wv_ref[...], preferred_element_type=jnp.float32)
    for h in range(HEADS):
        vh = v[:, h * DIM_HEAD:(h + 1) * DIM_HEAD]
        vt_ref[h * HW:(h + 1) * HW, 0:DIM_HEAD] = vh
        vt_ref[h * HW:(h + 1) * HW, DIM_HEAD:2 * DIM_HEAD] = jnp.concatenate(
            [vh[1:], vh[HW - 1:HW]], axis=0)

    x_b = x_ref[pl.ds(b * N, N), :]
    h_aw = _gelu_exact(jnp.dot(x_b, aW1_ref[...],
                               preferred_element_type=jnp.float32)
                       + ab1_ref[...])
    logits = jnp.dot(h_aw, aW2_ref[...],
                     preferred_element_type=jnp.float32) + ab2_ref[...]
    e = jnp.exp(logits)
    ci = lax.broadcasted_iota(jnp.int32, (HP, HP), 0) // N_POINTS
    cj = lax.broadcasted_iota(jnp.int32, (HP, HP), 1) // N_POINTS
    S = (ci == cj).astype(jnp.float32)
    attw = jnp.transpose(
        e / jnp.dot(e, S, preferred_element_type=jnp.float32))

    ni = lax.broadcasted_iota(jnp.int32, (N, B * N), 0)
    rsel = (ni // 16) * N + (ni % 16) * 4 + b
    Gsel = (rsel == lax.broadcasted_iota(jnp.int32, (N, B * N), 1)
            ).astype(jnp.float32)
    xo_b = jnp.dot(Gsel, xall, preferred_element_type=jnp.float32)
    h_off = _gelu_exact(jnp.dot(xo_b, oW1_ref[...],
                                preferred_element_type=jnp.float32)
                        + ob1_ref[...])
    t2 = jnp.transpose(jnp.tanh(
        jnp.dot(h_off, oW2_ref[...],
                preferred_element_type=jnp.float32) + ob2_ref[...]))
    pi = lax.broadcasted_iota(jnp.int32, (HP, 2 * HP), 0)
    pj = lax.broadcasted_iota(jnp.int32, (HP, 2 * HP), 1)
    gx = jnp.dot((pj == 2 * pi).astype(jnp.float32), t2,
                 preferred_element_type=jnp.float32)
    gy = jnp.dot((pj == 2 * pi + 1).astype(jnp.float32), t2,
                 preferred_element_type=jnp.float32)

    half = (GRID - 1) * 0.5
    ix = (gx + 1.0) * half
    iy = (gy + 1.0) * half
    ix0f = jnp.floor(ix)
    iy0f = jnp.floor(iy)
    wx1 = ix - ix0f
    wx0 = 1.0 - wx1
    wy1 = iy - iy0f
    wy0 = 1.0 - wy1
    ix0 = ix0f.astype(jnp.int32)
    iy0 = iy0f.astype(jnp.int32)
    iy1 = jnp.minimum(iy0 + 1, GRID - 1)

    hrow = lax.broadcasted_iota(jnp.int32, (HP, N), 0) // N_POINTS
    mapbase = (b * HEADS + hrow) * HW

    for c, (cy, cwy) in enumerate(((iy0, wy0), (iy1, wy1))):
        idx_ref[0, c] = mapbase + cy * GRID + ix0
        w_ref[0, c, 0] = attw * cwy * wx0
        w_ref[0, c, 1] = attw * cwy * wx1


def _sc_body(table, idx_hbm, w_hbm, out_hbm, idx_v, w2_v, w_v, G_v, acc_v,
             sem, gsem0, gsem1):
    gsems = (gsem0, gsem1)
    c = lax.axis_index("c")
    s = lax.axis_index("s")
    wid = c * 16 + s
    b = wid // HEADS
    h = wid % HEADS

    stage = []
    for corner in range(2):
        stage.append(pltpu.make_async_copy(
            idx_hbm.at[b, corner, pl.ds(h * N_POINTS, N_POINTS)],
            idx_v.at[pl.ds(corner * N_POINTS, N_POINTS)], sem))
        for y in range(2):
            blk = corner * 2 + y
            stage.append(pltpu.make_async_copy(
                w_hbm.at[b, corner, y, pl.ds(h * N_POINTS, N_POINTS)],
                w2_v.at[pl.ds(blk * N_POINTS, N_POINTS)], sem))
    for cp in stage:
        cp.start()
    for cp in stage:
        cp.wait()
    for j in range(4 * N_POINTS):
        for k in range(4):
            w_v[pl.ds(j * N + k * 16, 16)] = w2_v[j, k * 16:(k + 1) * 16]

    NR = 4
    ROUNDS = 16 // NR

    def fire(r, buf):
        cps = []
        for j in range(NR):
            cps.append(pltpu.make_async_copy(
                table.at[idx_v.at[r * NR + j]],
                G_v.at[pl.ds((buf * NR + j) * N, N)], gsems[buf]))
        for cp in cps:
            cp.start()
        return cps

    pend = fire(0, 0)
    for r in range(ROUNDS):
        nxt = fire(r + 1, (r + 1) % 2) if r + 1 < ROUNDS else None
        for cp in pend:
            cp.wait()
        buf = r % 2

        def nbody(n, _, r=r, buf=buf):
            nvec = jnp.full((16,), n, dtype=jnp.int32)
            accs = [None] * 4
            for j in range(NR):
                jj = r * NR + j
                cnr = jj // N_POINTS
                p = jj % N_POINTS
                wbA = plsc.load_gather(
                    w_v, [nvec + ((cnr * 2 + 0) * N_POINTS + p) * N])
                wbB = plsc.load_gather(
                    w_v, [nvec + ((cnr * 2 + 1) * N_POINTS + p) * N])
                row = (buf * NR + j) * N + n
                for k in range(4):
                    gA = G_v[row, k * 16:(k + 1) * 16]
                    gB = G_v[row, DIM_HEAD + k * 16:DIM_HEAD + (k + 1) * 16]
                    t = wbA * gA + wbB * gB
                    accs[k] = t if accs[k] is None else accs[k] + t
            for k in range(4):
                if r == 0:
                    acc_v[n, k * 16:(k + 1) * 16] = accs[k]
                else:
                    acc_v[n, k * 16:(k + 1) * 16] = (
                        acc_v[n, k * 16:(k + 1) * 16] + accs[k])
            return 0

        lax.fori_loop(0, N, nbody, 0)
        pend = nxt

    pltpu.sync_copy(acc_v, out_hbm.at[b, h])


def _stage3(s_ref, pW_ref, pb_ref, out_ref):
    acc = None
    for h in range(HEADS):
        t = jnp.dot(s_ref[0, h], pW_ref[h * DIM_HEAD:(h + 1) * DIM_HEAD],
                    preferred_element_type=jnp.float32)
        acc = t if acc is None else acc + t
    out_ref[0] = acc + pb_ref[...]


@jax.jit
def kernel(x, context, W_q, W_v, off_W1, off_b1, off_W2, off_b2,
           aw_W1, aw_b1, aw_W2, aw_b2, out_W, out_b):
    full = lambda *s: pl.BlockSpec(s, lambda b: (0,) * len(s))
    col = lambda v: v.reshape(-1, 1)

    vt, idx, w = pl.pallas_call(
        _stage1,
        grid=(B,),
        in_specs=[
            full(B * N, DIM),
            pl.BlockSpec((1, HW, DIM), lambda b: (b, 0, 0)),
            full(DIM, INNER),
            full(DIM, DIM),
            full(1, DIM),
            full(DIM, 2 * HP),
            full(1, 2 * HP),
            full(DIM, DIM),
            full(1, DIM),
            full(DIM, HP),
            full(1, HP),
        ],
        out_specs=[
            pl.BlockSpec((HEADS * HW, 2 * DIM_HEAD), lambda b: (b, 0)),
            pl.BlockSpec((1, 2, HP, N), lambda b: (b, 0, 0, 0)),
            pl.BlockSpec((1, 2, 2, HP, N), lambda b: (b, 0, 0, 0, 0)),
        ],
        out_shape=[
            jax.ShapeDtypeStruct((B * HEADS * HW, 2 * DIM_HEAD), jnp.float32),
            jax.ShapeDtypeStruct((B, 2, HP, N), jnp.int32),
            jax.ShapeDtypeStruct((B, 2, 2, HP, N), jnp.float32),
        ],
    )(x.reshape(B * N, DIM), context, W_v,
      off_W1, off_b1.reshape(1, -1), off_W2, off_b2.reshape(1, -1),
      aw_W1, aw_b1.reshape(1, -1), aw_W2, aw_b2.reshape(1, -1))

    mesh = plsc.VectorSubcoreMesh(core_axis_name="c", subcore_axis_name="s")
    sampled = pl.kernel(
        _sc_body,
        out_type=jax.ShapeDtypeStruct((B, HEADS, N, DIM_HEAD), jnp.float32),
        mesh=mesh,
        compiler_params=pltpu.CompilerParams(needs_layout_passes=False,
                                             use_tc_tiling_on_sc=True),
        scratch_types=[
            pltpu.VMEM((2 * N_POINTS, N), jnp.int32),
            pltpu.VMEM((4 * N_POINTS, N), jnp.float32),
            pltpu.VMEM((4 * N_POINTS * N,), jnp.float32),
            pltpu.VMEM((8 * N, 2 * DIM_HEAD), jnp.float32),
            pltpu.VMEM((N, DIM_HEAD), jnp.float32),
            pltpu.SemaphoreType.DMA,
            pltpu.SemaphoreType.DMA,
            pltpu.SemaphoreType.DMA,
        ],
    )(vt, idx, w)

    out = pl.pallas_call(
        _stage3,
        grid=(B,),
        in_specs=[
            pl.BlockSpec((1, HEADS, N, DIM_HEAD), lambda b: (b, 0, 0, 0)),
            full(INNER, DIM),
            full(1, DIM),
        ],
        out_specs=pl.BlockSpec((1, N, DIM), lambda b: (b, 0, 0)),
        out_shape=jax.ShapeDtypeStruct((B, N, DIM), jnp.float32),
    )(sampled, out_W, out_b.reshape(1, -1))
    return out
